# Initial kernel scaffold; baseline (speedup 1.0000x reference)
import jax
import jax.numpy as jnp
from jax.experimental import pallas as pl


def kernel(interaction_x, interaction_edge_index, similarity_x, similarity_edge_index, W_ic1, b_ic1, W_ic2, b_ic2, W_sc1, b_sc1, W_sc2, b_sc2, fc_W, fc_b):
    raise NotImplementedError("write your pallas kernel here")


# speedup vs baseline: 19.6960x; 19.6960x over previous
import functools

import jax
import jax.numpy as jnp
from jax import lax
from jax.experimental import pallas as pl
from jax.experimental.pallas import tpu as pltpu
from jax.experimental.pallas import tpu_sc as plsc

N = 10000
E = 320000
H = 128

NC = 2
NS = 16
NW = NC * NS
EPT = E // NW
K = 100
ST = 5
CPS = 20
NP = 10240
RPT = NP // NS

F32 = jnp.float32


def _mesh():
    return plsc.VectorSubcoreMesh(
        core_axis_name="c", subcore_axis_name="s",
        num_cores=NC, num_subcores=NS)


def _sc_deg(dsts):
    out_t = (jax.ShapeDtypeStruct((NC, NP, 16), F32),
             jax.ShapeDtypeStruct((NC, NP, 16), F32))

    @functools.partial(
        pl.kernel,
        out_type=out_t,
        mesh=_mesh(),
        scratch_types=[
            pltpu.VMEM((CPS, K), jnp.int32),
            pltpu.VMEM((K, 16), F32),
            pltpu.VMEM_SHARED((NP, 16), F32),
        ],
    )
    def k(dsts_hbm, oi_hbm, os_hbm, idx_v, ones_v, acc):
        c = lax.axis_index("c")
        s = lax.axis_index("s")
        wid = c * NS + s

        def _fill(val):
            def _f(i, carry):
                ones_v[i, :] = jnp.full((16,), val, F32)
                return carry
            lax.fori_loop(0, K, _f, 0)

        for g, out_hbm in enumerate((oi_hbm, os_hbm)):
            _fill(0.0)
            for j in range(RPT // K):
                pltpu.sync_copy(ones_v, acc.at[pl.ds(s * RPT + j * K, K)])
            pltpu.sync_copy(ones_v.at[pl.ds(0, RPT % K)],
                            acc.at[pl.ds(s * RPT + (RPT // K) * K, RPT % K)])
            plsc.subcore_barrier()
            _fill(1.0)

            for b in range(ST):
                pltpu.sync_copy(dsts_hbm.at[g, wid, b], idx_v)

                def _step(r, carry):
                    pltpu.sync_copy(ones_v, acc.at[idx_v.at[r]], add=True)
                    return carry
                lax.fori_loop(0, CPS, _step, 0)
            plsc.subcore_barrier()

            pltpu.sync_copy(acc.at[pl.ds(s * RPT, RPT)],
                            out_hbm.at[c, pl.ds(s * RPT, RPT)])

    return k(dsts)


def _sc_scatter(xs, src, dst):
    @functools.partial(
        pl.kernel,
        out_type=jax.ShapeDtypeStruct((NC, NP, H), F32),
        mesh=_mesh(),
        scratch_types=[
            pltpu.VMEM((CPS, K), jnp.int32),
            pltpu.VMEM((CPS, K), jnp.int32),
            pltpu.VMEM((K, H), F32),
            pltpu.VMEM((K, H), F32),
            pltpu.VMEM_SHARED((NP, H), F32),
            pltpu.SemaphoreType.DMA,
            pltpu.SemaphoreType.DMA,
        ],
    )
    def k(xs_hbm, src_hbm, dst_hbm, out_hbm,
          src_v, dst_v, rows0, rows1, acc, sem0, sem1):
        c = lax.axis_index("c")
        s = lax.axis_index("s")
        wid = c * NS + s

        def _zrow(i, carry):
            for j in range(H // 16):
                rows0[i, pl.ds(j * 16, 16)] = jnp.zeros((16,), F32)
                rows1[i, pl.ds(j * 16, 16)] = jnp.zeros((16,), F32)
            return carry
        lax.fori_loop(0, K, _zrow, 0)
        for j in range(RPT // (2 * K)):
            pltpu.sync_copy(rows0, acc.at[pl.ds(s * RPT + j * 2 * K, K)])
            pltpu.sync_copy(rows1, acc.at[pl.ds(s * RPT + j * 2 * K + K, K)])
        pltpu.sync_copy(rows0.at[pl.ds(0, RPT % (2 * K))],
                        acc.at[pl.ds(s * RPT + (RPT // (2 * K)) * 2 * K,
                                     RPT % (2 * K))])
        plsc.subcore_barrier()

        for b in range(ST):
            pltpu.sync_copy(src_hbm.at[wid, b], src_v)
            pltpu.sync_copy(dst_hbm.at[wid, b], dst_v)

            def _pair(i, carry):
                r = 2 * i
                d0 = pltpu.async_copy(xs_hbm.at[src_v.at[r]], rows0, sem0)
                d1 = pltpu.async_copy(xs_hbm.at[src_v.at[r + 1]], rows1, sem1)
                d0.wait()
                pltpu.sync_copy(rows0, acc.at[dst_v.at[r]], add=True)
                d1.wait()
                pltpu.sync_copy(rows1, acc.at[dst_v.at[r + 1]], add=True)
                return carry
            lax.fori_loop(0, CPS // 2, _pair, 0)
        plsc.subcore_barrier()

        pltpu.sync_copy(acc.at[pl.ds(s * RPT, RPT)],
                        out_hbm.at[c, pl.ds(s * RPT, RPT)])

    return k(xs, src, dst)


_R = 1000
_GRID = (N // _R,)


def _dinv_of(dp):
    return lax.rsqrt(1.0 + dp[0, :, 0] + dp[1, :, 0])[:, None]


def _row_spec():
    return pl.BlockSpec((_R, H), lambda i: (i, 0))


def _w_spec():
    return pl.BlockSpec((H, H), lambda i: (0, 0))


def _deg_spec():
    return pl.BlockSpec((NC, _R, 16), lambda i: (0, i, 0))


def _part_spec():
    return pl.BlockSpec((NC, _R, H), lambda i: (0, i, 0))


def _b_spec():
    return pl.BlockSpec((1, H), lambda i: (0, 0))


def _prep_body(x_ref, wt_ref, dp_ref, o_ref):
    dinv = _dinv_of(dp_ref[...])
    o_ref[...] = dinv * jnp.dot(x_ref[...], wt_ref[...],
                                preferred_element_type=F32)


def _tc_prep(x, wt, degp):
    return pl.pallas_call(
        _prep_body,
        grid=_GRID,
        in_specs=[_row_spec(), _w_spec(), _deg_spec()],
        out_specs=_row_spec(),
        out_shape=jax.ShapeDtypeStruct((N, H), F32),
    )(x, wt, degp)


def _mid_body(p_ref, xs_ref, dp_ref, b_ref, wt_ref, o_ref):
    dinv = _dinv_of(dp_ref[...])
    p = p_ref[...]
    h = jnp.maximum(dinv * (p[0] + p[1] + xs_ref[...]) + b_ref[...], 0.0)
    o_ref[...] = dinv * jnp.dot(h, wt_ref[...], preferred_element_type=F32)


def _tc_mid(parts, xs, degp, b, wt):
    return pl.pallas_call(
        _mid_body,
        grid=_GRID,
        in_specs=[_part_spec(), _row_spec(), _deg_spec(), _b_spec(),
                  _w_spec()],
        out_specs=_row_spec(),
        out_shape=jax.ShapeDtypeStruct((N, H), F32),
    )(parts, xs, degp, b, wt)


def _fin_body(pi_ref, xsi_ref, dpi_ref, bi_ref,
              ps_ref, xss_ref, dps_ref, bs_ref,
              fwi_ref, fws_ref, fb_ref, o_ref):
    dinv_i = _dinv_of(dpi_ref[...])
    pi = pi_ref[...]
    h_i = jnp.maximum(
        dinv_i * (pi[0] + pi[1] + xsi_ref[...]) + bi_ref[...], 0.0)
    dinv_s = _dinv_of(dps_ref[...])
    ps = ps_ref[...]
    h_s = jnp.maximum(
        dinv_s * (ps[0] + ps[1] + xss_ref[...]) + bs_ref[...], 0.0)
    o_ref[...] = (jnp.dot(h_i, fwi_ref[...], preferred_element_type=F32)
                  + jnp.dot(h_s, fws_ref[...], preferred_element_type=F32)
                  + fb_ref[...])


def _tc_fin(pi, xsi, dpi, bi, ps, xss, dps, bs, fwi, fws, fb):
    return pl.pallas_call(
        _fin_body,
        grid=_GRID,
        in_specs=[_part_spec(), _row_spec(), _deg_spec(), _b_spec(),
                  _part_spec(), _row_spec(), _deg_spec(), _b_spec(),
                  _w_spec(), _w_spec(), _b_spec()],
        out_specs=_row_spec(),
        out_shape=jax.ShapeDtypeStruct((N, H), F32),
    )(pi, xsi, dpi, bi, ps, xss, dps, bs, fwi, fws, fb)


def kernel(interaction_x, interaction_edge_index,
           similarity_x, similarity_edge_index,
           W_ic1, b_ic1, W_ic2, b_ic2,
           W_sc1, b_sc1, W_sc2, b_sc2, fc_W, fc_b):
    src_i = interaction_edge_index[0].reshape(NW, ST, CPS, K)
    dst_i = interaction_edge_index[1].reshape(NW, ST, CPS, K)
    src_s = similarity_edge_index[0].reshape(NW, ST, CPS, K)
    dst_s = similarity_edge_index[1].reshape(NW, ST, CPS, K)
    dsts = jnp.stack([dst_i, dst_s])

    degp_i, degp_s = _sc_deg(dsts)

    b1_i = b_ic1.reshape(1, H)
    b2_i = b_ic2.reshape(1, H)
    b1_s = b_sc1.reshape(1, H)
    b2_s = b_sc2.reshape(1, H)
    fwt = fc_W.T
    fwt_i = fwt[:H]
    fwt_s = fwt[H:]
    fb = fc_b.reshape(1, H)

    xs1_i = _tc_prep(interaction_x, W_ic1.T, degp_i)
    xs1_s = _tc_prep(similarity_x, W_sc1.T, degp_s)
    p1_i = _sc_scatter(xs1_i, src_i, dst_i)
    p1_s = _sc_scatter(xs1_s, src_s, dst_s)
    xs2_i = _tc_mid(p1_i, xs1_i, degp_i, b1_i, W_ic2.T)
    xs2_s = _tc_mid(p1_s, xs1_s, degp_s, b1_s, W_sc2.T)
    p2_i = _sc_scatter(xs2_i, src_i, dst_i)
    p2_s = _sc_scatter(xs2_s, src_s, dst_s)
    return _tc_fin(p2_i, xs2_i, degp_i, b2_i,
                   p2_s, xs2_s, degp_s, b2_s,
                   fwt_i, fwt_s, fb)
